# baseline (speedup 1.0000x reference)
import functools

import jax
import jax.numpy as jnp
from jax import lax
from jax.experimental import pallas as pl
from jax.experimental.pallas import tpu as pltpu
from jax.experimental.pallas import tpu_sc as plsc

B = 16384
D = 32
U_ROWS = 1000000

_info = plsc.get_sparse_core_info()
NC, NS, L = _info.num_cores, _info.num_subcores, _info.num_lanes
NW = NC * NS
BPW = B // NW
CHUNK = 128
NCHUNK = BPW // CHUNK


def _mf_body(bu_hbm, bi_hbm, p_hbm, q_hbm, out_hbm,
             idx_u, idx_i, p_v, q_v, out_v, sem):
    wid = lax.axis_index("s") * NC + lax.axis_index("c")
    base = wid * BPW
    pltpu.sync_copy(bu_hbm.at[pl.ds(base, BPW)], idx_u)
    pltpu.sync_copy(bi_hbm.at[pl.ds(base, BPW)], idx_i)

    handles = []
    for c in range(NCHUNK):
        sl = pl.ds(c * CHUNK, CHUNK)
        handles.append(pltpu.async_copy(p_hbm.at[idx_u.at[sl]], p_v.at[sl], sem))
        handles.append(pltpu.async_copy(q_hbm.at[idx_i.at[sl]], q_v.at[sl], sem))
    for h in handles:
        h.wait()

    lane = lax.iota(jnp.int32, L)
    perms = [lane ^ o for o in (8, 4, 2, 1)]

    def permute(v, perm):
        return lax.gather(
            v, perm[:, None],
            lax.GatherDimensionNumbers(
                offset_dims=(), collapsed_slice_dims=(0,), start_index_map=(0,)),
            slice_sizes=(1,),
            mode=lax.GatherScatterMode.PROMISE_IN_BOUNDS)

    def group(g, carry):
        acc = jnp.zeros((L,), jnp.float32)
        for r in range(L):
            b = g * L + r
            pw = plsc.bitcast(p_v[b, :], jnp.bfloat16)
            qw = plsc.bitcast(q_v[b, :], jnp.bfloat16)
            p0, p1 = plsc.unpack(pw, format=plsc.PackFormat.INTERLEAVED)
            q0, q1 = plsc.unpack(qw, format=plsc.PackFormat.INTERLEAVED)
            prod = p0 * q0 + p1 * q1
            for perm in perms:
                prod = prod + permute(prod, perm)
            acc = jnp.where(lane == r, prod, acc)
        out_v[pl.ds(g * L, L)] = acc
        return carry

    lax.fori_loop(0, BPW // L, group, 0)
    pltpu.sync_copy(out_v, out_hbm.at[pl.ds(base, BPW)])


@functools.partial(
    pl.kernel,
    mesh=plsc.VectorSubcoreMesh(core_axis_name="c", subcore_axis_name="s"),
    out_type=jax.ShapeDtypeStruct((B,), jnp.float32),
    scratch_types=[
        pltpu.VMEM((BPW,), jnp.int32),
        pltpu.VMEM((BPW,), jnp.int32),
        pltpu.VMEM((BPW, D // 2), jnp.int32),
        pltpu.VMEM((BPW, D // 2), jnp.int32),
        pltpu.VMEM((BPW,), jnp.float32),
        pltpu.SemaphoreType.DMA,
    ],
    compiler_params=pltpu.CompilerParams(
        use_tc_tiling_on_sc=False, needs_layout_passes=False),
)
def _mf_sc(bu, bi, p, q, out, *scratch):
    _mf_body(bu, bi, p, q, out, *scratch)


def kernel(b_users, b_items, P, Q):
    pw = jax.lax.bitcast_convert_type(
        P.astype(jnp.bfloat16).reshape(U_ROWS, D // 2, 2), jnp.int32)
    qw = jax.lax.bitcast_convert_type(
        Q.astype(jnp.bfloat16).reshape(U_ROWS, D // 2, 2), jnp.int32)
    out = _mf_sc(b_users.astype(jnp.int32), b_items.astype(jnp.int32),
                 pw, qw)
    return out[:, None]


# speedup vs baseline: 9.8138x; 9.8138x over previous
import functools

import jax
import jax.numpy as jnp
from jax import lax
from jax.experimental import pallas as pl
from jax.experimental.pallas import tpu as pltpu
from jax.experimental.pallas import tpu_sc as plsc

B = 16384
D = 32

_info = plsc.get_sparse_core_info()
NC, NS, L = _info.num_cores, _info.num_subcores, _info.num_lanes
NW = NC * NS
BPW = B // NW
RING = 8


def _mf_body(bu_hbm, bi_hbm, pt_hbm, qt_hbm, out_hbm,
             idx_u, idx_i, pslab, qslab, out_v, semp, semq):
    wid = lax.axis_index("s") * NC + lax.axis_index("c")
    base = wid * BPW
    pltpu.sync_copy(bu_hbm.at[pl.ds(base, BPW)], idx_u)
    pltpu.sync_copy(bi_hbm.at[pl.ds(base, BPW)], idx_i)

    lane = lax.iota(jnp.int32, L)
    dlo = lane
    dhi = lane + L
    perms = [lane ^ o for o in (8, 4, 2, 1)]

    def permute(v, perm):
        return lax.gather(
            v, perm[:, None],
            lax.GatherDimensionNumbers(
                offset_dims=(), collapsed_slice_dims=(0,), start_index_map=(0,)),
            slice_sizes=(1,),
            mode=lax.GatherScatterMode.PROMISE_IN_BOUNDS)

    def read_idx(ref, j):
        vec = ref[pl.ds((j >> 4) * L, L)]
        sel = permute(vec, jnp.full((L,), j & (L - 1), jnp.int32))
        return sel[0]

    def fire(j, slot):
        r_u = read_idx(idx_u, j)
        r_i = read_idx(idx_i, j)
        au = pl.multiple_of((r_u >> 7) << 7, 128)
        ai = pl.multiple_of((r_i >> 7) << 7, 128)
        pltpu.async_copy(pt_hbm.at[:, pl.ds(au, 128)], pslab.at[slot], semp)
        pltpu.async_copy(qt_hbm.at[:, pl.ds(ai, 128)], qslab.at[slot], semq)

    def drain(slot):
        pltpu.make_async_copy(
            pt_hbm.at[:, pl.ds(0, 128)], pslab.at[slot], semp).wait()
        pltpu.make_async_copy(
            qt_hbm.at[:, pl.ds(0, 128)], qslab.at[slot], semq).wait()

    for j in range(RING):
        fire(j, j)

    def body(j, acc):
        slot = j & (RING - 1)
        drain(slot)
        r_u = read_idx(idx_u, j)
        r_i = read_idx(idx_i, j)
        lu = jnp.full((L,), r_u & 127, jnp.int32)
        li = jnp.full((L,), r_i & 127, jnp.int32)
        p0 = plsc.load_gather(pslab.at[slot], [dlo, lu])
        p1 = plsc.load_gather(pslab.at[slot], [dhi, lu])
        q0 = plsc.load_gather(qslab.at[slot], [dlo, li])
        q1 = plsc.load_gather(qslab.at[slot], [dhi, li])
        prod = p0 * q0 + p1 * q1
        for perm in perms:
            prod = prod + permute(prod, perm)
        acc = jnp.where(lane == (j & (L - 1)), prod, acc)

        @pl.when((j & (L - 1)) == (L - 1))
        def _store():
            out_v[pl.ds((j >> 4) * L, L)] = acc

        acc = jnp.where((j & (L - 1)) == (L - 1), jnp.zeros((L,), jnp.float32),
                        acc)

        @pl.when(j + RING < BPW)
        def _fire():
            fire(j + RING, slot)

        return acc

    lax.fori_loop(0, BPW, body, jnp.zeros((L,), jnp.float32))
    pltpu.sync_copy(out_v, out_hbm.at[pl.ds(base, BPW)])


@functools.partial(
    pl.kernel,
    mesh=plsc.VectorSubcoreMesh(core_axis_name="c", subcore_axis_name="s"),
    out_type=jax.ShapeDtypeStruct((B,), jnp.float32),
    scratch_types=[
        pltpu.VMEM((BPW,), jnp.int32),
        pltpu.VMEM((BPW,), jnp.int32),
        pltpu.VMEM((RING, D, 128), jnp.float32),
        pltpu.VMEM((RING, D, 128), jnp.float32),
        pltpu.VMEM((BPW,), jnp.float32),
        pltpu.SemaphoreType.DMA,
        pltpu.SemaphoreType.DMA,
    ],
    compiler_params=pltpu.CompilerParams(
        use_tc_tiling_on_sc=True, needs_layout_passes=False),
)
def _mf_sc(bu, bi, pt, qt, out, *scratch):
    _mf_body(bu, bi, pt, qt, out, *scratch)


def kernel(b_users, b_items, P, Q):
    out = _mf_sc(b_users.astype(jnp.int32), b_items.astype(jnp.int32),
                 P.T, Q.T)
    return out[:, None]
